# baseline (speedup 1.0000x reference)
import jax
import jax.numpy as jnp
from jax.experimental import pallas as pl
from jax.experimental.pallas import tpu as pltpu


_NCHUNK = 64


def _body(idx_ref, k_hbm, v_hbm, xk_hbm, xv_hbm, ok_hbm, ov_hbm, sem_bulk, sem_r):
    n = k_hbm.shape[0]
    c = n // _NCHUNK
    copies = []
    for i in range(_NCHUNK):
        sl = pl.ds(i * c, c)
        copies.append(pltpu.make_async_copy(k_hbm.at[sl], ok_hbm.at[sl], sem_bulk))
        copies.append(pltpu.make_async_copy(v_hbm.at[sl], ov_hbm.at[sl], sem_bulk))
    for cp in copies:
        cp.start()
    for cp in copies:
        cp.wait()
    idx = idx_ref[0]
    rk = pltpu.make_async_copy(xk_hbm, ok_hbm.at[:, pl.ds(idx, 1), :], sem_r)
    rv = pltpu.make_async_copy(xv_hbm, ov_hbm.at[:, pl.ds(idx, 1), :], sem_r)
    rk.start()
    rv.start()
    rk.wait()
    rv.wait()


def kernel(k_cache, v_cache, xk, xv, layer_past_len):
    B, Hkv, S, D = k_cache.shape
    idx = jnp.asarray(layer_past_len, jnp.int32).reshape((1,))
    k3 = k_cache.reshape(B * Hkv, S, D)
    v3 = v_cache.reshape(B * Hkv, S, D)
    xk3 = xk.reshape(B * Hkv, 1, D)
    xv3 = xv.reshape(B * Hkv, 1, D)
    grid_spec = pltpu.PrefetchScalarGridSpec(
        num_scalar_prefetch=1,
        grid=(1,),
        in_specs=[pl.BlockSpec(memory_space=pltpu.MemorySpace.HBM)] * 4,
        out_specs=[pl.BlockSpec(memory_space=pltpu.MemorySpace.HBM)] * 2,
        scratch_shapes=[pltpu.SemaphoreType.DMA] * 2,
    )
    ok, ov = pl.pallas_call(
        _body,
        grid_spec=grid_spec,
        out_shape=(
            jax.ShapeDtypeStruct(k3.shape, k3.dtype),
            jax.ShapeDtypeStruct(v3.shape, v3.dtype),
        ),
    )(idx, k3, v3, xk3, xv3)
    return ok.reshape(B, Hkv, S, D), ov.reshape(B, Hkv, S, D)


# speedup vs baseline: 48.6783x; 48.6783x over previous
import jax
import jax.numpy as jnp
from jax.experimental import pallas as pl
from jax.experimental.pallas import tpu as pltpu


_G = 4


def _body(idx_ref, k_ref, v_ref, xk_ref, xv_ref, ok_ref, ov_ref):
    idx = idx_ref[0]
    ok_ref[...] = k_ref[...]
    ov_ref[...] = v_ref[...]
    ok_ref[:, pl.ds(idx, 1), :] = xk_ref[...]
    ov_ref[:, pl.ds(idx, 1), :] = xv_ref[...]


def kernel(k_cache, v_cache, xk, xv, layer_past_len):
    B, Hkv, S, D = k_cache.shape
    N = B * Hkv
    idx = jnp.asarray(layer_past_len, jnp.int32).reshape((1,))
    k3 = k_cache.reshape(N, S, D)
    v3 = v_cache.reshape(N, S, D)
    xk3 = xk.reshape(N, 1, D)
    xv3 = xv.reshape(N, 1, D)
    cache_spec = pl.BlockSpec((_G, S, D), lambda i, idx_ref: (i, 0, 0))
    x_spec = pl.BlockSpec((_G, 1, D), lambda i, idx_ref: (i, 0, 0))
    grid_spec = pltpu.PrefetchScalarGridSpec(
        num_scalar_prefetch=1,
        grid=(N // _G,),
        in_specs=[cache_spec, cache_spec, x_spec, x_spec],
        out_specs=[cache_spec, cache_spec],
    )
    ok, ov = pl.pallas_call(
        _body,
        grid_spec=grid_spec,
        out_shape=(
            jax.ShapeDtypeStruct(k3.shape, k3.dtype),
            jax.ShapeDtypeStruct(v3.shape, v3.dtype),
        ),
    )(idx, k3, v3, xk3, xv3)
    return ok.reshape(B, Hkv, S, D), ov.reshape(B, Hkv, S, D)
